# baseline (speedup 1.0000x reference)
import functools

import jax
import jax.numpy as jnp
from jax import lax
from jax.experimental import pallas as pl
from jax.experimental.pallas import tpu as pltpu
from jax.experimental.pallas import tpu_sc as plsc

_N = 10000
_E = 320000
_IN_CH = 128
_HID = 64
_NUM_LAYERS = 5
_NUM_GRAPHS = 64
_NCLS = 2
_BN_EPS = 1e-5

_NC = 2
_NS = 16
_NW = _NC * _NS

_C = 128
_TCH = 80
_NBUF = 4
_NGRP = _TCH // _NBUF
_PER_TILE = _C * _TCH
_EP = _NW * _PER_TILE
_PAD = _EP - _E
_NACC = 10112
_ZROWS = _NACC // _NS

_mesh = plsc.VectorSubcoreMesh(core_axis_name="c", subcore_axis_name="s")


@functools.partial(
    pl.kernel,
    mesh=_mesh,
    compiler_params=pltpu.CompilerParams(use_tc_tiling_on_sc=False),
    out_type=jax.ShapeDtypeStruct((_NC, _NACC, _HID), jnp.float32),
    scratch_types=[
        pltpu.VMEM((_TCH, _C), jnp.int32),
        pltpu.VMEM((_TCH, _C), jnp.int32),
        [pltpu.VMEM((_C, _HID), jnp.float32) for _ in range(_NBUF)],
        [pltpu.SemaphoreType.DMA for _ in range(_NBUF)],
        pltpu.VMEM_SHARED((_NACC, _HID), jnp.float32),
    ],
)
def _edge_agg(q_hbm, src_hbm, dst_hbm, zeros_hbm, out_hbm,
              sidx, didx, rows, sems, acc):
    c = lax.axis_index("c")
    s = lax.axis_index("s")
    tile = c * _NS + s
    pltpu.sync_copy(zeros_hbm.at[pl.ds(s * _ZROWS, _ZROWS)],
                    acc.at[pl.ds(s * _ZROWS, _ZROWS)])
    pltpu.sync_copy(src_hbm.at[tile], sidx)
    pltpu.sync_copy(dst_hbm.at[tile], didx)
    plsc.subcore_barrier()

    for b in range(_NBUF):
        pltpu.async_copy(q_hbm.at[sidx.at[b]], rows[b], sems[b])

    def group(g, carry):
        for b in range(_NBUF):
            j = g * _NBUF + b
            pltpu.make_async_copy(q_hbm.at[sidx.at[j]], rows[b], sems[b]).wait()
            pltpu.sync_copy(rows[b], acc.at[didx.at[j]], add=True)
            pltpu.async_copy(q_hbm.at[sidx.at[j + _NBUF]], rows[b], sems[b])
        return carry

    lax.fori_loop(0, _NGRP - 1, group, 0)
    for b in range(_NBUF):
        j = (_NGRP - 1) * _NBUF + b
        pltpu.make_async_copy(q_hbm.at[sidx.at[j]], rows[b], sems[b]).wait()
        pltpu.sync_copy(rows[b], acc.at[didx.at[j]], add=True)
    plsc.subcore_barrier()
    pltpu.sync_copy(acc.at[pl.ds(s * _ZROWS, _ZROWS)],
                    out_hbm.at[c, pl.ds(s * _ZROWS, _ZROWS)])


def _tc_pre_body(x_ref, w1_ref, lin0_ref, q_ref, nout_ref):
    x = x_ref[...]
    q_ref[...] = jnp.dot(x, w1_ref[...], preferred_element_type=jnp.float32, precision=lax.Precision.HIGHEST)
    nout_ref[...] = jnp.dot(x, lin0_ref[...], preferred_element_type=jnp.float32, precision=lax.Precision.HIGHEST)


_tc_pre = pl.pallas_call(
    _tc_pre_body,
    out_shape=[
        jax.ShapeDtypeStruct((_N, _HID), jnp.float32),
        jax.ShapeDtypeStruct((_N, _NCLS), jnp.float32),
    ],
)


def _layer_core(q_ref, parts_ref, b1_ref, w2_ref, b2_ref, gamma_ref, beta_ref):
    z = q_ref[...] + parts_ref[0, 0:_N, :] + parts_ref[1, 0:_N, :] + b1_ref[...]
    z = jnp.maximum(z, 0.0)
    z = jnp.dot(z, w2_ref[...], preferred_element_type=jnp.float32, precision=lax.Precision.HIGHEST) + b2_ref[...]
    mean = jnp.mean(z, axis=0, keepdims=True)
    zc = z - mean
    var = jnp.mean(zc * zc, axis=0, keepdims=True)
    h = zc * lax.rsqrt(var + _BN_EPS) * gamma_ref[...] + beta_ref[...]
    return jnp.maximum(h, 0.0)


def _tc_layer_body(q_ref, parts_ref, b1_ref, w2_ref, b2_ref, gamma_ref,
                   beta_ref, w1n_ref, lin_ref, nin_ref, qn_ref, nout_ref):
    h = _layer_core(q_ref, parts_ref, b1_ref, w2_ref, b2_ref, gamma_ref, beta_ref)
    qn_ref[...] = jnp.dot(h, w1n_ref[...], preferred_element_type=jnp.float32, precision=lax.Precision.HIGHEST)
    nout_ref[...] = nin_ref[...] + jnp.dot(
        h, lin_ref[...], preferred_element_type=jnp.float32, precision=lax.Precision.HIGHEST)


_tc_layer = pl.pallas_call(
    _tc_layer_body,
    out_shape=[
        jax.ShapeDtypeStruct((_N, _HID), jnp.float32),
        jax.ShapeDtypeStruct((_N, _NCLS), jnp.float32),
    ],
)


def _tc_last_body(q_ref, parts_ref, b1_ref, w2_ref, b2_ref, gamma_ref,
                  beta_ref, lin_ref, nin_ref, batch_ref, linb_ref, out_ref):
    h = _layer_core(q_ref, parts_ref, b1_ref, w2_ref, b2_ref, gamma_ref, beta_ref)
    nout = nin_ref[...] + jnp.dot(h, lin_ref[...], preferred_element_type=jnp.float32, precision=lax.Precision.HIGHEST)
    gids = lax.broadcasted_iota(jnp.int32, (_NUM_GRAPHS, _N), 0)
    onehot = (batch_ref[...] == gids).astype(jnp.float32)
    out_ref[...] = jnp.dot(
        onehot, nout, preferred_element_type=jnp.float32, precision=lax.Precision.HIGHEST) + linb_ref[...]


_tc_last = pl.pallas_call(
    _tc_last_body,
    out_shape=jax.ShapeDtypeStruct((_NUM_GRAPHS, _NCLS), jnp.float32),
)


def kernel(x, edge_index, batch, params):
    layers = params["layers"]
    lin_w = params["lin_w"]
    lin_b = params["lin_b"]

    src = edge_index[0].astype(jnp.int32)
    dst = edge_index[1].astype(jnp.int32)
    src_p = jnp.concatenate(
        [src, jnp.zeros((_PAD,), jnp.int32)]).reshape(_NW, _TCH, _C)
    dst_p = jnp.concatenate(
        [dst, jnp.full((_PAD,), _N, jnp.int32)]).reshape(_NW, _TCH, _C)
    zeros_acc = jnp.zeros((_NACC, _HID), jnp.float32)
    batch2d = batch.astype(jnp.int32).reshape(1, _N)

    q, nout = _tc_pre(x, layers[0]["W1"], lin_w[0:_IN_CH])
    out = None
    for i in range(_NUM_LAYERS):
        p = layers[i]
        parts = _edge_agg(q, src_p, dst_p, zeros_acc)
        lin_sl = lax.slice(lin_w, (_IN_CH + i * _HID, 0),
                           (_IN_CH + (i + 1) * _HID, _NCLS))
        common = (p["b1"].reshape(1, -1), p["W2"], p["b2"].reshape(1, -1),
                  p["gamma"].reshape(1, -1), p["beta"].reshape(1, -1))
        if i < _NUM_LAYERS - 1:
            q, nout = _tc_layer(q, parts, *common, layers[i + 1]["W1"],
                                lin_sl, nout)
        else:
            out = _tc_last(q, parts, *common, lin_sl, nout, batch2d,
                           lin_b.reshape(1, -1))
    return out


# speedup vs baseline: 2.1795x; 2.1795x over previous
import functools

import jax
import jax.numpy as jnp
from jax import lax
from jax.experimental import pallas as pl
from jax.experimental.pallas import tpu as pltpu
from jax.experimental.pallas import tpu_sc as plsc

_N = 10000
_E = 320000
_IN_CH = 128
_HID = 64
_NUM_LAYERS = 5
_NUM_GRAPHS = 64
_NCLS = 2
_BN_EPS = 1e-5

_NC = 2
_NS = 16
_NW = _NC * _NS

_C = 128
_CH = _HID // _NC
_TCH = 160
_NBUF = 4
_NGRP = _TCH // _NBUF
_PER_TILE = _C * _TCH
_EP = _NS * _PER_TILE
_PAD = _EP - _E
_NACC = 10112
_ZROWS = _NACC // _NS

_mesh = plsc.VectorSubcoreMesh(core_axis_name="c", subcore_axis_name="s")


@functools.partial(
    pl.kernel,
    mesh=_mesh,
    compiler_params=pltpu.CompilerParams(use_tc_tiling_on_sc=False),
    out_type=jax.ShapeDtypeStruct((_NACC, _HID), jnp.float32),
    scratch_types=[
        pltpu.VMEM((_TCH, _C), jnp.int32),
        pltpu.VMEM((_TCH, _C), jnp.int32),
        [pltpu.VMEM((_C, _CH), jnp.float32) for _ in range(_NBUF)],
        [pltpu.SemaphoreType.DMA for _ in range(_NBUF)],
        pltpu.VMEM_SHARED((_NACC, _CH), jnp.float32),
        pltpu.VMEM_SHARED((_NACC, _CH), jnp.float32),
    ],
)
def _edge_agg(q_hbm, src_hbm, dst_hbm, zeros_hbm, out_hbm,
              sidx, didx, rows, sems, acc, qsh):
    c = lax.axis_index("c")
    s = lax.axis_index("s")
    col = c * _CH
    pltpu.sync_copy(zeros_hbm.at[pl.ds(s * _ZROWS, _ZROWS)],
                    acc.at[pl.ds(s * _ZROWS, _ZROWS)])
    pltpu.sync_copy(q_hbm.at[pl.ds(s * _ZROWS, _ZROWS), pl.ds(col, _CH)],
                    qsh.at[pl.ds(s * _ZROWS, _ZROWS)])
    pltpu.sync_copy(src_hbm.at[s], sidx)
    pltpu.sync_copy(dst_hbm.at[s], didx)
    plsc.subcore_barrier()

    for b in range(_NBUF):
        pltpu.async_copy(qsh.at[sidx.at[b]], rows[b], sems[b])

    def group(g, carry):
        for b in range(_NBUF):
            j = g * _NBUF + b
            pltpu.make_async_copy(qsh.at[sidx.at[j]], rows[b], sems[b]).wait()
            pltpu.sync_copy(rows[b], acc.at[didx.at[j]], add=True)
            pltpu.async_copy(qsh.at[sidx.at[j + _NBUF]], rows[b], sems[b])
        return carry

    lax.fori_loop(0, _NGRP - 1, group, 0)
    for b in range(_NBUF):
        j = (_NGRP - 1) * _NBUF + b
        pltpu.make_async_copy(qsh.at[sidx.at[j]], rows[b], sems[b]).wait()
        pltpu.sync_copy(rows[b], acc.at[didx.at[j]], add=True)
    plsc.subcore_barrier()
    pltpu.sync_copy(acc.at[pl.ds(s * _ZROWS, _ZROWS)],
                    out_hbm.at[pl.ds(s * _ZROWS, _ZROWS), pl.ds(col, _CH)])


def _tc_pre_body(x_ref, w1_ref, lin0_ref, q_ref, nout_ref):
    x = x_ref[...]
    q_ref[0:_N, :] = jnp.dot(x, w1_ref[...], preferred_element_type=jnp.float32, precision=lax.Precision.HIGHEST)
    nout_ref[...] = jnp.dot(x, lin0_ref[...], preferred_element_type=jnp.float32, precision=lax.Precision.HIGHEST)


_tc_pre = pl.pallas_call(
    _tc_pre_body,
    out_shape=[
        jax.ShapeDtypeStruct((_NACC, _HID), jnp.float32),
        jax.ShapeDtypeStruct((_N, _NCLS), jnp.float32),
    ],
)


def _layer_core(q_ref, parts_ref, b1_ref, w2_ref, b2_ref, gamma_ref, beta_ref):
    z = q_ref[0:_N, :] + parts_ref[0:_N, :] + b1_ref[...]
    z = jnp.maximum(z, 0.0)
    z = jnp.dot(z, w2_ref[...], preferred_element_type=jnp.float32, precision=lax.Precision.HIGHEST) + b2_ref[...]
    mean = jnp.mean(z, axis=0, keepdims=True)
    zc = z - mean
    var = jnp.mean(zc * zc, axis=0, keepdims=True)
    h = zc * lax.rsqrt(var + _BN_EPS) * gamma_ref[...] + beta_ref[...]
    return jnp.maximum(h, 0.0)


def _tc_layer_body(q_ref, parts_ref, b1_ref, w2_ref, b2_ref, gamma_ref,
                   beta_ref, w1n_ref, lin_ref, nin_ref, qn_ref, nout_ref):
    h = _layer_core(q_ref, parts_ref, b1_ref, w2_ref, b2_ref, gamma_ref, beta_ref)
    qn_ref[0:_N, :] = jnp.dot(h, w1n_ref[...], preferred_element_type=jnp.float32, precision=lax.Precision.HIGHEST)
    nout_ref[...] = nin_ref[...] + jnp.dot(
        h, lin_ref[...], preferred_element_type=jnp.float32, precision=lax.Precision.HIGHEST)


_tc_layer = pl.pallas_call(
    _tc_layer_body,
    out_shape=[
        jax.ShapeDtypeStruct((_NACC, _HID), jnp.float32),
        jax.ShapeDtypeStruct((_N, _NCLS), jnp.float32),
    ],
)


def _tc_last_body(q_ref, parts_ref, b1_ref, w2_ref, b2_ref, gamma_ref,
                  beta_ref, lin_ref, nin_ref, batch_ref, linb_ref, out_ref):
    h = _layer_core(q_ref, parts_ref, b1_ref, w2_ref, b2_ref, gamma_ref, beta_ref)
    nout = nin_ref[...] + jnp.dot(h, lin_ref[...], preferred_element_type=jnp.float32, precision=lax.Precision.HIGHEST)
    gids = lax.broadcasted_iota(jnp.int32, (_NUM_GRAPHS, _N), 0)
    onehot = (batch_ref[...] == gids).astype(jnp.float32)
    out_ref[...] = jnp.dot(
        onehot, nout, preferred_element_type=jnp.float32, precision=lax.Precision.HIGHEST) + linb_ref[...]


_tc_last = pl.pallas_call(
    _tc_last_body,
    out_shape=jax.ShapeDtypeStruct((_NUM_GRAPHS, _NCLS), jnp.float32),
)


def kernel(x, edge_index, batch, params):
    layers = params["layers"]
    lin_w = params["lin_w"]
    lin_b = params["lin_b"]

    src = edge_index[0].astype(jnp.int32)
    dst = edge_index[1].astype(jnp.int32)
    src_p = jnp.concatenate(
        [src, jnp.zeros((_PAD,), jnp.int32)]).reshape(_NS, _TCH, _C)
    dst_p = jnp.concatenate(
        [dst, jnp.full((_PAD,), _N, jnp.int32)]).reshape(_NS, _TCH, _C)
    zeros_acc = jnp.zeros((_NACC, _CH), jnp.float32)
    batch2d = batch.astype(jnp.int32).reshape(1, _N)

    q, nout = _tc_pre(x, layers[0]["W1"], lin_w[0:_IN_CH])
    out = None
    for i in range(_NUM_LAYERS):
        p = layers[i]
        parts = _edge_agg(q, src_p, dst_p, zeros_acc)
        lin_sl = lax.slice(lin_w, (_IN_CH + i * _HID, 0),
                           (_IN_CH + (i + 1) * _HID, _NCLS))
        common = (p["b1"].reshape(1, -1), p["W2"], p["b2"].reshape(1, -1),
                  p["gamma"].reshape(1, -1), p["beta"].reshape(1, -1))
        if i < _NUM_LAYERS - 1:
            q, nout = _tc_layer(q, parts, *common, layers[i + 1]["W1"],
                                lin_sl, nout)
        else:
            out = _tc_last(q, parts, *common, lin_sl, nout, batch2d,
                           lin_b.reshape(1, -1))
    return out
